# baseline (speedup 1.0000x reference)
import functools
import math

import jax
import jax.numpy as jnp
from jax import lax
from jax.experimental import pallas as pl
from jax.experimental.pallas import tpu as pltpu
from jax.experimental.pallas import tpu_sc as plsc

LATENT = 64
N_NODES = 10000
IN_DIM = 12
T_STEPS = 3
OUT_DIM = 12
N_LAYERS = 4
MAX_RADIUS_EDGES = 400000

NPAD = 10240
NPAD2 = NPAD // 2
PAD_ROW = N_NODES
N_TILES = 32
CH = 64
IDXW = 128
ROW_BLK = 2000


def _ln(x, g, b, eps=1e-5):
    m = jnp.mean(x, axis=-1, keepdims=True)
    v = jnp.mean((x - m) ** 2, axis=-1, keepdims=True)
    return (x - m) / jnp.sqrt(v + eps) * g + b


def _rsqrt16(v):
    i = lax.bitcast_convert_type(v, jnp.int32)
    y = lax.bitcast_convert_type(jnp.int32(0x5F3759DF) - (i >> 1), jnp.float32)
    for _ in range(3):
        y = y * (1.5 - 0.5 * v * y * y)
    return y


_GDN = lax.GatherDimensionNumbers(offset_dims=(), collapsed_slice_dims=(0,),
                                  start_index_map=(0,))


def _shuffle16(v, idx):
    return lax.gather(v, idx[:, None], _GDN, (1,),
                      mode=lax.GatherScatterMode.PROMISE_IN_BOUNDS)


def _vsum_splat(v):
    lanes = lax.iota(jnp.int32, 16)
    for k in (8, 4, 2, 1):
        v = v + _shuffle16(v, lanes ^ k)
    return v


def _edge_ln_chunk(a_v, b_v, m_v, g_v, be_v, pv_lo, pv_hi):
    gs = [g_v[pl.ds(k * 16, 16)] for k in range(4)]
    bes = [be_v[pl.ds(k * 16, 16)] for k in range(4)]
    for e in range(CH):
        ys = [a_v[e, pl.ds(k * 16, 16)] + b_v[e, pl.ds(64 + k * 16, 16)]
              for k in range(4)]
        tot = (ys[0] + ys[1]) + (ys[2] + ys[3])
        mu = _vsum_splat(tot) * (1.0 / 64.0)
        cs = [y - mu for y in ys]
        sq = (cs[0] * cs[0] + cs[1] * cs[1]) + (cs[2] * cs[2] + cs[3] * cs[3])
        var = _vsum_splat(sq) * (1.0 / 64.0) + 1e-5
        rstd = _rsqrt16(var)
        for k in range(4):
            m = cs[k] * rstd * gs[k] + bes[k]
            m_v[e, pl.ds(k * 16, 16)] = m * pv_lo
            m_v[e, pl.ds(64 + k * 16, 16)] = m * pv_hi


@functools.partial(jax.jit, static_argnames=("n_chunks",))
def _sc_edge_contact(AB, src3, dst3, dsth3, par3, zeros_fold, g, beta, *,
                     n_chunks):
    mesh = plsc.VectorSubcoreMesh(core_axis_name="c", subcore_axis_name="s")

    @functools.partial(
        pl.kernel, mesh=mesh,
        out_type=jax.ShapeDtypeStruct((2, NPAD2, 2 * LATENT), jnp.float32),
        scratch_types=[
            pltpu.VMEM((n_chunks, CH), jnp.int32),
            pltpu.VMEM((n_chunks, CH), jnp.int32),
            pltpu.VMEM((n_chunks, CH), jnp.int32),
            pltpu.VMEM((n_chunks, 16), jnp.float32),
            pltpu.VMEM((CH, 2 * LATENT), jnp.float32),
            pltpu.VMEM((CH, 2 * LATENT), jnp.float32),
            pltpu.VMEM((CH, 2 * LATENT), jnp.float32),
            pltpu.VMEM((LATENT,), jnp.float32),
            pltpu.VMEM((LATENT,), jnp.float32),
            pltpu.VMEM_SHARED((NPAD2, 2 * LATENT), jnp.float32),
            pltpu.SemaphoreType.DMA,
            pltpu.SemaphoreType.DMA,
        ],
    )
    def k(AB_hbm, src_hbm, dst_hbm, dsth_hbm, par_hbm, z_hbm, g_hbm, be_hbm,
          out_hbm, src_v, dst_v, dsth_v, par_v, a_v, b_v, m_v, g_v, be_v,
          agg_sh, sem_a, sem_b):
        c = lax.axis_index("c")
        s = lax.axis_index("s")
        wid = c * 16 + s
        rows_per_sub = NPAD2 // 16
        r0 = s * rows_per_sub
        pltpu.sync_copy(z_hbm.at[pl.ds(r0, rows_per_sub)],
                        agg_sh.at[pl.ds(r0, rows_per_sub)])
        pltpu.sync_copy(g_hbm, g_v)
        pltpu.sync_copy(be_hbm, be_v)
        pltpu.sync_copy(src_hbm.at[wid], src_v)
        pltpu.sync_copy(dst_hbm.at[wid], dst_v)
        pltpu.sync_copy(dsth_hbm.at[wid], dsth_v)
        pltpu.sync_copy(par_hbm.at[wid], par_v)
        plsc.subcore_barrier()

        def chunk(ci, carry):
            cp_a = pltpu.async_copy(AB_hbm.at[src_v.at[ci]], a_v, sem_a)
            cp_b = pltpu.async_copy(AB_hbm.at[dst_v.at[ci]], b_v, sem_b)
            pv_hi = par_v[ci, pl.ds(0, 16)]
            pv_lo = 1.0 - pv_hi
            cp_a.wait()
            cp_b.wait()
            _edge_ln_chunk(a_v, b_v, m_v, g_v, be_v, pv_lo, pv_hi)
            pltpu.sync_copy(m_v, agg_sh.at[dsth_v.at[ci]], add=True)
            return carry

        lax.fori_loop(0, n_chunks, chunk, 0)
        plsc.subcore_barrier()
        pltpu.sync_copy(agg_sh.at[pl.ds(r0, rows_per_sub)],
                        out_hbm.at[c, pl.ds(r0, rows_per_sub)])

    return k(AB, src3, dst3, dsth3, par3, zeros_fold, g, beta)


def _pad_edges(src, dst, n_chunks_tot):
    e = src.shape[0]
    n_chunks = n_chunks_tot // N_TILES
    tot = n_chunks_tot * CH
    par = (dst & 1).astype(jnp.int32)
    order = jnp.argsort(par, stable=True)
    ssrc = src[order].astype(jnp.int32)
    sdst = dst[order].astype(jnp.int32)
    n_even = e - jnp.sum(par)
    c0 = (n_even + CH - 1) // CH
    off_odd = c0 * CH
    k = jnp.arange(e)
    slot = jnp.where(k < n_even, k, off_odd + (k - n_even))
    chunk_of_slot = jnp.arange(tot, dtype=jnp.int32) // CH
    odd_region = chunk_of_slot >= c0
    dst_def = jnp.where(odd_region, PAD_ROW + 1, PAD_ROW).astype(jnp.int32)
    srcp = jnp.full((tot,), PAD_ROW, jnp.int32).at[slot].set(ssrc)
    dstp = dst_def.at[slot].set(sdst)
    src3 = srcp.reshape(N_TILES, n_chunks, CH)
    dst3 = dstp.reshape(N_TILES, n_chunks, CH)
    chunk_par = odd_region.reshape(-1, CH)[:, :1].astype(jnp.float32)
    par3 = jnp.broadcast_to(chunk_par, (n_chunks_tot, 16)
                            ).reshape(N_TILES, n_chunks, 16)
    return src3, dst3, dst3 >> 1, par3, n_chunks


def _contact_sc(h, src3, dst3, dsth3, par3, n_chunks, p):
    AB = jnp.concatenate([h @ p['eW'][:, :LATENT].T,
                          h @ p['eW'][:, LATENT:].T + p['eb']], axis=1)
    ABp = jnp.concatenate(
        [AB, jnp.zeros((NPAD - N_NODES, 2 * LATENT), jnp.float32)], axis=0)
    zeros_fold = jnp.zeros((NPAD2, 2 * LATENT), jnp.float32)
    parts = _sc_edge_contact(ABp, src3, dst3, dsth3, par3, zeros_fold,
                             p['eg'], p['ebeta'], n_chunks=n_chunks)
    agg = (parts[0] + parts[1]).reshape(NPAD, LATENT)[:N_NODES]
    u = h @ p['nW'][:, :LATENT].T + agg @ p['nW'][:, LATENT:].T + p['nb']
    u = _ln(u, p['ng'], p['nbeta'])
    return h + u


def _head_body(ht_ref, hr_ref, apw_ref, apb_ref, apg_ref, apbe_ref,
               d1w_ref, d1b_ref, d2w_ref, d2b_ref, dg_ref, dbe_ref, out_ref):
    hcat = jnp.concatenate([ht_ref[...], hr_ref[...]], axis=1)
    h1 = lax.dot_general(hcat, apw_ref[...], (((1,), (1,)), ((), ())),
                         preferred_element_type=jnp.float32) + apb_ref[...]
    h1 = _ln(h1, apg_ref[...], apbe_ref[...])
    h2 = lax.dot_general(h1, d1w_ref[...], (((1,), (1,)), ((), ())),
                         preferred_element_type=jnp.float32) + d1b_ref[...]
    h2 = jnp.maximum(h2, 0.0)
    h3 = lax.dot_general(h2, d2w_ref[...], (((1,), (1,)), ((), ())),
                         preferred_element_type=jnp.float32) + d2b_ref[...]
    out_ref[...] = _ln(h3, dg_ref[...], dbe_ref[...])


def _head(h_topo, h_radius, p):
    n = h_topo.shape[0]
    grid = n // ROW_BLK
    row = lambda i: (i, 0)
    full = lambda i: (0, 0)
    w2 = lambda a: a.reshape(1, -1)
    return pl.pallas_call(
        _head_body,
        grid=(grid,),
        in_specs=[
            pl.BlockSpec((ROW_BLK, LATENT), row),
            pl.BlockSpec((ROW_BLK, LATENT), row),
            pl.BlockSpec((LATENT, 2 * LATENT), full),
            pl.BlockSpec((1, LATENT), full),
            pl.BlockSpec((1, LATENT), full),
            pl.BlockSpec((1, LATENT), full),
            pl.BlockSpec((LATENT, LATENT), full),
            pl.BlockSpec((1, LATENT), full),
            pl.BlockSpec((OUT_DIM, LATENT), full),
            pl.BlockSpec((1, OUT_DIM), full),
            pl.BlockSpec((1, OUT_DIM), full),
            pl.BlockSpec((1, OUT_DIM), full),
        ],
        out_specs=pl.BlockSpec((ROW_BLK, OUT_DIM), row),
        out_shape=jax.ShapeDtypeStruct((n, OUT_DIM), jnp.float32),
    )(h_topo, h_radius, p['ap_W'], w2(p['ap_b']), w2(p['ap_g']), w2(p['ap_be']),
      p['d_W1'], w2(p['d_b1']), p['d_W2'], w2(p['d_b2']), w2(p['d_g']), w2(p['d_be']))


def _contact_jnp(h, src, dst, p, valid=None):
    A = h @ p['eW'][:, :LATENT].T
    B = h @ p['eW'][:, LATENT:].T + p['eb']
    m = _ln(A[src] + B[dst], p['eg'], p['ebeta'])
    if valid is not None:
        m = jnp.where(valid[:, None], m, 0.0)
    agg = jax.ops.segment_sum(m, dst, num_segments=h.shape[0])
    u = h @ p['nW'][:, :LATENT].T + agg @ p['nW'][:, LATENT:].T + p['nb']
    u = _ln(u, p['ng'], p['nbeta'])
    return h + u


def _temporal(x, p):
    xs = jnp.transpose(x, (0, 2, 1))
    N = xs.shape[0]
    inp = xs
    h = None
    for l in range(3):
        h = jnp.zeros((N, LATENT), xs.dtype)
        c = jnp.zeros((N, LATENT), xs.dtype)
        outs = []
        for t in range(inp.shape[1]):
            g = (inp[:, t, :] @ p['W_ih%d' % l].T + p['b_ih%d' % l]
                 + h @ p['W_hh%d' % l].T + p['b_hh%d' % l])
            i_, f_, gg, o_ = jnp.split(g, 4, axis=1)
            i_ = jax.nn.sigmoid(i_); f_ = jax.nn.sigmoid(f_)
            gg = jnp.tanh(gg); o_ = jax.nn.sigmoid(o_)
            c = f_ * c + i_ * gg
            h = o_ * jnp.tanh(c)
            outs.append(h)
        inp = jnp.stack(outs, axis=1)
    return _ln(h @ p['fc_W'].T + p['fc_b'], p['fc_g'], p['fc_be'])


def _radius_edges(coords, r):
    N = coords.shape[0]
    r2 = r * r
    d2 = jnp.sum((coords[:, None, :] - coords[None, :, :]) ** 2, axis=-1)
    idx = jnp.arange(N)
    mask = (d2 <= r2) & (idx[:, None] != idx[None, :])
    src, dst = jnp.nonzero(mask, size=MAX_RADIUS_EDGES, fill_value=0)
    count = jnp.sum(mask)
    pos = jnp.arange(MAX_RADIUS_EDGES)
    valid = (pos < count) | ((pos == 0) & (count == 0))
    return src.astype(jnp.int32), dst.astype(jnp.int32), valid


def kernel(x, topo_edge_index, params):
    r_src, r_dst, r_valid = _radius_edges(x[:, :3, -1], 2.0)
    h0 = _temporal(x, params['temporal'])
    n_ch_tot = ((160000 // CH + 1 + N_TILES - 1) // N_TILES) * N_TILES
    t_s3, t_d3, t_dh3, t_p3, t_chunks = _pad_edges(
        topo_edge_index[0], topo_edge_index[1], n_ch_tot)
    h_topo = h0
    for i in range(N_LAYERS):
        h_topo = _contact_sc(h_topo, t_s3, t_d3, t_dh3, t_p3, t_chunks,
                             params['topo'][i])
    h_radius = _contact_jnp(h0, r_src, r_dst, params['radius'], r_valid)
    return _head(h_topo, h_radius, params)
